# baseline (speedup 1.0000x reference)
import numpy as np
import jax
import jax.numpy as jnp
from jax import lax
from jax.experimental import pallas as pl
from jax.experimental.pallas import tpu as pltpu
from jax.experimental.pallas import tpu_sc as plsc

N = 10000
E = 320000
C = 4
CD = 3

NC, NS, L = 2, 16, 16
NW = NC * NS
EPW = E // NW

CB2 = 80
CB5 = 80
NP = 10240
DCH = 80
DRND = NP // (DCH * NS)
CB7 = 80

EB3 = 2000
EB4 = 512


def _radial_mats():
    ML = np.zeros((128, 48), np.float32)
    MR = np.zeros((128, 48), np.float32)
    for g in range(CD):
        for c in range(C):
            for f in range(C):
                ML[3 * c + g, g * 16 + c * 4 + f] = 1.0
                MR[3 * f + g, g * 16 + c * 4 + f] = 1.0
    return ML, MR


_ML_NP, _MR_NP = _radial_mats()


def _k1_body(h_ref, wq_ref, wkv_ref, bq_ref, bkv_ref, hq_ref, hkv_ref):
    h = h_ref[...]
    hq_ref[...] = jnp.dot(h, wq_ref[...], preferred_element_type=jnp.float32) + bq_ref[...]
    hkv_ref[...] = jnp.dot(h, wkv_ref[...], preferred_element_type=jnp.float32) + bkv_ref[...]


def _k2_body(row_hbm, col_hbm, ctab_hbm, hq_hbm, hkv_hbm,
             cd_hbm, hqr_hbm, hkvc_hbm,
             rowv, colv, ca, cb, cdv, qv, kvv, sem):
    c = lax.axis_index("c")
    s = lax.axis_index("s")
    wid = s * NC + c
    base = wid * EPW

    zero = jnp.zeros((L,), jnp.float32)

    def z8(j, carry):
        for kk in range(8):
            cdv[j, pl.ds(kk * L, L)] = zero
        return carry

    lax.fori_loop(0, CB2, z8, 0)

    def chunk(i, carry):
        off = base + i * CB2
        pltpu.sync_copy(row_hbm.at[pl.ds(off, CB2)], rowv)
        pltpu.sync_copy(col_hbm.at[pl.ds(off, CB2)], colv)
        cp1 = pltpu.async_copy(ctab_hbm.at[rowv], ca, sem)
        cp2 = pltpu.async_copy(ctab_hbm.at[colv], cb, sem)
        cp3 = pltpu.async_copy(hq_hbm.at[rowv], qv, sem)
        cp4 = pltpu.async_copy(hkv_hbm.at[colv], kvv, sem)
        cp1.wait()
        cp2.wait()
        cp3.wait()
        cp4.wait()

        def sub(j, carry2):
            cdv[j, pl.ds(0, L)] = ca[j, pl.ds(0, L)] - cb[j, pl.ds(0, L)]
            return carry2

        lax.fori_loop(0, CB2, sub, 0)
        pltpu.sync_copy(cdv, cd_hbm.at[pl.ds(off, CB2)])
        pltpu.sync_copy(qv, hqr_hbm.at[pl.ds(off, CB2)])
        pltpu.sync_copy(kvv, hkvc_hbm.at[pl.ds(off, CB2)])
        return carry

    lax.fori_loop(0, EPW // CB2, chunk, 0)


def _k3_body(cd_ref, ml_ref, mr_ref, cdrad_ref, ss_ref):
    cd = cd_ref[...]
    lm = jnp.dot(cd, ml_ref[...], preferred_element_type=jnp.float32)
    rm = jnp.dot(cd, mr_ref[...], preferred_element_type=jnp.float32)
    rad = (lm[:, 0:16] * rm[:, 0:16] + lm[:, 16:32] * rm[:, 16:32]
           + lm[:, 32:48] * rm[:, 32:48])
    cdrad_ref[...] = jnp.concatenate(
        [cd[:, 0:16], rad, cd[:, 32:128]], axis=1)

    @pl.when(pl.program_id(0) == 0)
    def _():
        ss_ref[...] = jnp.zeros_like(ss_ref)

    ss_ref[...] += jnp.sum(rad * rad, axis=0, keepdims=True)


def _k4_body(cdrad_ref, ea_ref, hqr_ref, hkvc_ref, ss_ref,
             wrk_ref, wrv_ref, wek_ref, wev_ref, wc1_ref, bc1_ref, wcb_ref,
             exv_ref, tr_ref, ex_ref):
    inv = 1.0 / jnp.maximum(jnp.sqrt(ss_ref[...]), 1e-12)
    cdrad = cdrad_ref[...]
    radn = cdrad[:, 16:32] * inv
    ea = ea_ref[...]
    k = (hkvc_ref[:, :128]
         + jnp.dot(radn, wrk_ref[...], preferred_element_type=jnp.float32)
         + jnp.dot(ea, wek_ref[...], preferred_element_type=jnp.float32))
    v = (hkvc_ref[:, 128:]
         + jnp.dot(radn, wrv_ref[...], preferred_element_type=jnp.float32)
         + jnp.dot(ea, wev_ref[...], preferred_element_type=jnp.float32))
    alpha = jnp.sum(hqr_ref[...] * k, axis=1, keepdims=True)
    ex = jnp.exp(alpha)
    exv_ref[...] = ex * v
    u = jnp.dot(v, wc1_ref[...], preferred_element_type=jnp.float32) + bc1_ref[...]
    su = u * jax.nn.sigmoid(u)
    cmb = jnp.dot(su, wcb_ref[...], preferred_element_type=jnp.float32)
    tr = cdrad * (ex * cmb)
    lane = lax.broadcasted_iota(jnp.int32, tr.shape, 1)
    tr = jnp.where(lane >= 3 * C, 0.0, tr)
    tr = jnp.where(lane == 15, ex, tr)
    tr_ref[...] = tr
    ex_ref[...] = ex[:, 0]


def _k5_body(row_hbm, x_hbm, z_hbm, p_hbm, agg, idxv, dbuf):
    c = lax.axis_index("c")
    s = lax.axis_index("s")
    wid = s * NC + c

    def zinit(k, carry):
        r = (k * NS + s) * DCH
        pltpu.sync_copy(z_hbm.at[pl.ds(r, DCH)], dbuf)
        pltpu.sync_copy(dbuf, agg.at[pl.ds(r, DCH)])
        return carry

    lax.fori_loop(0, DRND, zinit, 0)
    plsc.subcore_barrier()

    base = wid * EPW

    def chunk(i, carry):
        off = base + i * CB5
        pltpu.sync_copy(row_hbm.at[pl.ds(off, CB5)], idxv)
        pltpu.sync_copy(x_hbm.at[pl.ds(off, CB5)], dbuf)
        pltpu.sync_copy(dbuf, agg.at[idxv], add=True)
        return carry

    lax.fori_loop(0, EPW // CB5, chunk, 0)
    plsc.subcore_barrier()

    def dump(k, carry):
        r = (k * NS + s) * DCH
        pltpu.sync_copy(agg.at[pl.ds(r, DCH)], dbuf)
        pltpu.sync_copy(dbuf, p_hbm.at[pl.ds(c * NP + r, DCH)])
        return carry

    lax.fori_loop(0, DRND, dump, 0)


def _k6_body(h_ref, c16_ref, pu_ref, pc_ref, hout_ref, cout_ref):
    aggc = pc_ref[0:N, 0:16] + pc_ref[NP:NP + N, 0:16]
    den = aggc[:, 15:16]
    rden = jnp.where(den != 0.0, 1.0 / den, 0.0)
    hout_ref[...] = h_ref[...] + (pu_ref[0:N, :] + pu_ref[NP:NP + N, :]) * rden
    cout_ref[...] = c16_ref[...] + aggc * rden


def _k7_body(row_hbm, ex_hbm, den_hbm, att_hbm, denv, idxv, exv, av):
    c = lax.axis_index("c")
    s = lax.axis_index("s")
    wid = s * NC + c
    base = wid * EPW
    pltpu.sync_copy(den_hbm, denv)

    def chunk(i, carry):
        off = base + i * CB7
        pltpu.sync_copy(row_hbm.at[pl.ds(off, CB7)], idxv)
        pltpu.sync_copy(ex_hbm.at[pl.ds(off, CB7)], exv)

        def j16(j, carry2):
            eidx = idxv[pl.ds(j * L, L)]
            den = plsc.load_gather(denv, [eidx])
            av[pl.ds(j * L, L)] = exv[pl.ds(j * L, L)] / den
            return carry2

        lax.fori_loop(0, CB7 // L, j16, 0)
        pltpu.sync_copy(av, att_hbm.at[pl.ds(off, CB7)])
        return carry

    lax.fori_loop(0, EPW // CB7, chunk, 0)


_SC_MESH = dict(core_axis_name="c", subcore_axis_name="s",
                num_cores=NC, num_subcores=NS)


def kernel(h, edge_index, coord, edge_attr, Wq, bq, Wkv, bkv, Wc1, bc1, Wc2):
    f32 = jnp.float32
    row = edge_index[0]
    col = edge_index[1]
    ctab = jnp.pad(coord.reshape(N, C * CD), ((0, 0), (0, 128 - C * CD)))
    coord16 = ctab[:, :16]

    Wk = Wkv[0::2]
    Wv = Wkv[1::2]
    bk = bkv[0::2]
    bv = bkv[1::2]
    WqT = Wq.T
    WkvT = jnp.concatenate([Wk[:, 16:144].T, Wv[:, 16:144].T], axis=1)
    bkv1 = jnp.concatenate([bk, bv])[None, :]
    bq1 = bq[None, :]
    WrkT, WekT = Wk[:, :16].T, Wk[:, 144:].T
    WrvT, WevT = Wv[:, :16].T, Wv[:, 144:].T
    Wc1T = Wc1.T
    bc11 = bc1[None, :]
    WcB = jnp.zeros((128, 128), f32)
    for cc in range(C):
        for dd in range(CD):
            WcB = WcB.at[:, 3 * cc + dd].set(Wc2[cc, :])
    ML = jnp.asarray(_ML_NP)
    MR = jnp.asarray(_MR_NP)

    hq, hkv = pl.pallas_call(
        _k1_body,
        out_shape=[jax.ShapeDtypeStruct((N, 128), f32),
                   jax.ShapeDtypeStruct((N, 256), f32)],
    )(h, WqT, WkvT, bq1, bkv1)

    k2 = pl.kernel(
        _k2_body,
        out_type=[jax.ShapeDtypeStruct((E, 128), f32),
                  jax.ShapeDtypeStruct((E, 128), f32),
                  jax.ShapeDtypeStruct((E, 256), f32)],
        mesh=plsc.VectorSubcoreMesh(**_SC_MESH),
        compiler_params=pltpu.CompilerParams(needs_layout_passes=False),
        scratch_types=[pltpu.VMEM((CB2,), jnp.int32),
                       pltpu.VMEM((CB2,), jnp.int32),
                       pltpu.VMEM((CB2, 128), f32),
                       pltpu.VMEM((CB2, 128), f32),
                       pltpu.VMEM((CB2, 128), f32),
                       pltpu.VMEM((CB2, 128), f32),
                       pltpu.VMEM((CB2, 256), f32),
                       pltpu.SemaphoreType.DMA],
    )
    cd128, hqr, hkvc = k2(row, col, ctab, hq, hkv)

    cdrad, ss = pl.pallas_call(
        _k3_body,
        grid=(E // EB3,),
        in_specs=[pl.BlockSpec((EB3, 128), lambda i: (i, 0)),
                  pl.BlockSpec((128, 48), lambda i: (0, 0)),
                  pl.BlockSpec((128, 48), lambda i: (0, 0))],
        out_specs=[pl.BlockSpec((EB3, 128), lambda i: (i, 0)),
                   pl.BlockSpec((1, 16), lambda i: (0, 0))],
        out_shape=[jax.ShapeDtypeStruct((E, 128), f32),
                   jax.ShapeDtypeStruct((1, 16), f32)],
    )(cd128, ML, MR)

    exv, trans, ex1d = pl.pallas_call(
        _k4_body,
        grid=(E // EB4,),
        in_specs=[pl.BlockSpec((EB4, 128), lambda i: (i, 0)),
                  pl.BlockSpec((EB4, 16), lambda i: (i, 0)),
                  pl.BlockSpec((EB4, 128), lambda i: (i, 0)),
                  pl.BlockSpec((EB4, 256), lambda i: (i, 0)),
                  pl.BlockSpec((1, 16), lambda i: (0, 0)),
                  pl.BlockSpec((16, 128), lambda i: (0, 0)),
                  pl.BlockSpec((16, 128), lambda i: (0, 0)),
                  pl.BlockSpec((16, 128), lambda i: (0, 0)),
                  pl.BlockSpec((16, 128), lambda i: (0, 0)),
                  pl.BlockSpec((128, 128), lambda i: (0, 0)),
                  pl.BlockSpec((1, 128), lambda i: (0, 0)),
                  pl.BlockSpec((128, 128), lambda i: (0, 0))],
        out_specs=[pl.BlockSpec((EB4, 128), lambda i: (i, 0)),
                   pl.BlockSpec((EB4, 128), lambda i: (i, 0)),
                   pl.BlockSpec((EB4,), lambda i: (i,))],
        out_shape=[jax.ShapeDtypeStruct((E, 128), f32),
                   jax.ShapeDtypeStruct((E, 128), f32),
                   jax.ShapeDtypeStruct((E,), f32)],
    )(cdrad, edge_attr, hqr, hkvc, ss,
      WrkT, WrvT, WekT, WevT, Wc1T, bc11, WcB)

    zu = jnp.zeros((NP, 128), f32)

    def make_k5():
        return pl.kernel(
            _k5_body,
            out_type=[jax.ShapeDtypeStruct((2 * NP, 128), f32)],
            mesh=plsc.VectorSubcoreMesh(**_SC_MESH),
            compiler_params=pltpu.CompilerParams(needs_layout_passes=False),
            scratch_types=[pltpu.VMEM_SHARED((NP, 128), f32),
                           pltpu.VMEM((CB5,), jnp.int32),
                           pltpu.VMEM((CB5, 128), f32)],
        )

    (pu,) = make_k5()(row, exv, zu)
    (pc,) = make_k5()(row, trans, zu)

    h_out, c16o = pl.pallas_call(
        _k6_body,
        out_shape=[jax.ShapeDtypeStruct((N, 128), f32),
                   jax.ShapeDtypeStruct((N, 16), f32)],
    )(h, coord16, pu, pc)

    den1d = pc[0:N, 15] + pc[NP:NP + N, 15]
    k7 = pl.kernel(
        _k7_body,
        out_type=[jax.ShapeDtypeStruct((E,), f32)],
        mesh=plsc.VectorSubcoreMesh(**_SC_MESH),
        compiler_params=pltpu.CompilerParams(needs_layout_passes=False),
        scratch_types=[pltpu.VMEM((N,), f32),
                       pltpu.VMEM((CB7,), jnp.int32),
                       pltpu.VMEM((CB7,), f32),
                       pltpu.VMEM((CB7,), f32)],
    )
    (att,) = k7(row, ex1d, den1d)

    coord_out = c16o[:, :12].reshape(N, C, CD)
    return h_out, coord_out, att


# speedup vs baseline: 1.1474x; 1.1474x over previous
import numpy as np
import jax
import jax.numpy as jnp
from jax import lax
from jax.experimental import pallas as pl
from jax.experimental.pallas import tpu as pltpu
from jax.experimental.pallas import tpu_sc as plsc

N = 10000
E = 320000
C = 4
CD = 3

NC, NS, L = 2, 16, 16
NW = NC * NS
EPW = E // NW

CB2 = 80
CB5 = 80
NP = 10240
DCH = 80
DRND = NP // (DCH * NS)
CB7 = 80

EB3 = 2000
EB4 = 2000


def _radial_mats():
    ML = np.zeros((128, 48), np.float32)
    MR = np.zeros((128, 48), np.float32)
    for g in range(CD):
        for c in range(C):
            for f in range(C):
                ML[3 * c + g, g * 16 + c * 4 + f] = 1.0
                MR[3 * f + g, g * 16 + c * 4 + f] = 1.0
    return ML, MR


_ML_NP, _MR_NP = _radial_mats()


def _k1_body(h_ref, wq_ref, wkv_ref, bq_ref, bkv_ref, hq_ref, hkv_ref):
    h = h_ref[...]
    hq_ref[...] = jnp.dot(h, wq_ref[...], preferred_element_type=jnp.float32) + bq_ref[...]
    hkv_ref[...] = jnp.dot(h, wkv_ref[...], preferred_element_type=jnp.float32) + bkv_ref[...]


def _k2_body(row_hbm, col_hbm, ctab_hbm, hq_hbm, hkv_hbm,
             cd_hbm, hqr_hbm, hkvc_hbm,
             rowv, colv, ca, cb, cdv, qv, kvv, sem):
    c = lax.axis_index("c")
    s = lax.axis_index("s")
    wid = s * NC + c
    base = wid * EPW

    zero = jnp.zeros((L,), jnp.float32)

    def z8(j, carry):
        for kk in range(8):
            cdv[j, pl.ds(kk * L, L)] = zero
        return carry

    lax.fori_loop(0, CB2, z8, 0)

    def chunk(i, carry):
        off = base + i * CB2
        pltpu.sync_copy(row_hbm.at[pl.ds(off, CB2)], rowv)
        pltpu.sync_copy(col_hbm.at[pl.ds(off, CB2)], colv)
        cp1 = pltpu.async_copy(ctab_hbm.at[rowv], ca, sem)
        cp2 = pltpu.async_copy(ctab_hbm.at[colv], cb, sem)
        cp3 = pltpu.async_copy(hq_hbm.at[rowv], qv, sem)
        cp4 = pltpu.async_copy(hkv_hbm.at[colv], kvv, sem)
        cp1.wait()
        cp2.wait()
        cp3.wait()
        cp4.wait()

        def sub(j, carry2):
            cdv[j, pl.ds(0, L)] = ca[j, pl.ds(0, L)] - cb[j, pl.ds(0, L)]
            return carry2

        lax.fori_loop(0, CB2, sub, 0)
        pltpu.sync_copy(cdv, cd_hbm.at[pl.ds(off, CB2)])
        pltpu.sync_copy(qv, hqr_hbm.at[pl.ds(off, CB2)])
        pltpu.sync_copy(kvv, hkvc_hbm.at[pl.ds(off, CB2)])
        return carry

    lax.fori_loop(0, EPW // CB2, chunk, 0)


def _k3_body(cd_ref, ml_ref, mr_ref, cdrad_ref, ss_ref):
    cd = cd_ref[...]
    lm = jnp.dot(cd, ml_ref[...], preferred_element_type=jnp.float32)
    rm = jnp.dot(cd, mr_ref[...], preferred_element_type=jnp.float32)
    rad = (lm[:, 0:16] * rm[:, 0:16] + lm[:, 16:32] * rm[:, 16:32]
           + lm[:, 32:48] * rm[:, 32:48])
    cdrad_ref[...] = jnp.concatenate(
        [cd[:, 0:16], rad, cd[:, 32:128]], axis=1)

    @pl.when(pl.program_id(0) == 0)
    def _():
        ss_ref[...] = jnp.zeros_like(ss_ref)

    ss_ref[...] += jnp.sum(rad * rad, axis=0, keepdims=True)


def _k4_body(cdrad_ref, ea_ref, hqr_ref, hkvc_ref, ss_ref,
             wrk_ref, wrv_ref, wek_ref, wev_ref, wc1_ref, bc1_ref, wcb_ref,
             exv_ref, tr_ref):
    inv = 1.0 / jnp.maximum(jnp.sqrt(ss_ref[...]), 1e-12)
    cdrad = cdrad_ref[...]
    radn = cdrad[:, 16:32] * inv
    ea = ea_ref[...]
    k = (hkvc_ref[:, :128]
         + jnp.dot(radn, wrk_ref[...], preferred_element_type=jnp.float32)
         + jnp.dot(ea, wek_ref[...], preferred_element_type=jnp.float32))
    v = (hkvc_ref[:, 128:]
         + jnp.dot(radn, wrv_ref[...], preferred_element_type=jnp.float32)
         + jnp.dot(ea, wev_ref[...], preferred_element_type=jnp.float32))
    alpha = jnp.sum(hqr_ref[...] * k, axis=1, keepdims=True)
    ex = jnp.exp(alpha)
    exv_ref[...] = ex * v
    u = jnp.dot(v, wc1_ref[...], preferred_element_type=jnp.float32) + bc1_ref[...]
    su = u * jax.nn.sigmoid(u)
    cmb = jnp.dot(su, wcb_ref[...], preferred_element_type=jnp.float32)
    tr = cdrad * (ex * cmb)
    lane = lax.broadcasted_iota(jnp.int32, tr.shape, 1)
    tr = jnp.where(lane >= 3 * C, 0.0, tr)
    tr = jnp.where(lane == 15, ex, tr)
    tr_ref[...] = tr


def _k5_body(row_hbm, x_hbm, z_hbm, p_hbm, agg, idxv, dbuf):
    c = lax.axis_index("c")
    s = lax.axis_index("s")
    wid = s * NC + c

    def zinit(k, carry):
        r = (k * NS + s) * DCH
        pltpu.sync_copy(z_hbm.at[pl.ds(r, DCH)], dbuf)
        pltpu.sync_copy(dbuf, agg.at[pl.ds(r, DCH)])
        return carry

    lax.fori_loop(0, DRND, zinit, 0)
    plsc.subcore_barrier()

    base = wid * EPW

    def chunk(i, carry):
        off = base + i * CB5
        pltpu.sync_copy(row_hbm.at[pl.ds(off, CB5)], idxv)
        pltpu.sync_copy(x_hbm.at[pl.ds(off, CB5)], dbuf)
        pltpu.sync_copy(dbuf, agg.at[idxv], add=True)
        return carry

    lax.fori_loop(0, EPW // CB5, chunk, 0)
    plsc.subcore_barrier()

    def dump(k, carry):
        r = (k * NS + s) * DCH
        pltpu.sync_copy(agg.at[pl.ds(r, DCH)], dbuf)
        pltpu.sync_copy(dbuf, p_hbm.at[pl.ds(c * NP + r, DCH)])
        return carry

    lax.fori_loop(0, DRND, dump, 0)


def _k6_body(h_ref, c16_ref, pu_ref, pc_ref, hout_ref, cout_ref):
    aggc = pc_ref[0:N, 0:16] + pc_ref[NP:NP + N, 0:16]
    den = aggc[:, 15:16]
    rden = jnp.where(den != 0.0, 1.0 / den, 0.0)
    hout_ref[...] = h_ref[...] + (pu_ref[0:N, :] + pu_ref[NP:NP + N, :]) * rden
    cout_ref[...] = c16_ref[...] + aggc * rden


def _k7_body(row_hbm, tr_hbm, den_hbm, att_hbm, denv, idxv, trv, av):
    c = lax.axis_index("c")
    s = lax.axis_index("s")
    wid = s * NC + c
    base = wid * EPW
    l15 = jnp.full((L,), 15, jnp.int32)
    pltpu.sync_copy(den_hbm, denv)

    def chunk(i, carry):
        off = base + i * CB7
        pltpu.sync_copy(row_hbm.at[pl.ds(off, CB7)], idxv)
        pltpu.sync_copy(tr_hbm.at[pl.ds(off, CB7)], trv)

        def j16(j, carry2):
            eidx = idxv[pl.ds(j * L, L)]
            den = plsc.load_gather(denv, [eidx])
            ridx = j * L + lax.iota(jnp.int32, L)
            ex = plsc.load_gather(trv, [ridx, l15])
            av[pl.ds(j * L, L)] = ex / den
            return carry2

        lax.fori_loop(0, CB7 // L, j16, 0)
        pltpu.sync_copy(av, att_hbm.at[pl.ds(off, CB7)])
        return carry

    lax.fori_loop(0, EPW // CB7, chunk, 0)


_SC_MESH = dict(core_axis_name="c", subcore_axis_name="s",
                num_cores=NC, num_subcores=NS)


def kernel(h, edge_index, coord, edge_attr, Wq, bq, Wkv, bkv, Wc1, bc1, Wc2):
    f32 = jnp.float32
    row = edge_index[0]
    col = edge_index[1]
    ctab = jnp.pad(coord.reshape(N, C * CD), ((0, 0), (0, 128 - C * CD)))
    coord16 = ctab[:, :16]

    Wk = Wkv[0::2]
    Wv = Wkv[1::2]
    bk = bkv[0::2]
    bv = bkv[1::2]
    WqT = Wq.T
    WkvT = jnp.concatenate([Wk[:, 16:144].T, Wv[:, 16:144].T], axis=1)
    bkv1 = jnp.concatenate([bk, bv])[None, :]
    bq1 = bq[None, :]
    WrkT, WekT = Wk[:, :16].T, Wk[:, 144:].T
    WrvT, WevT = Wv[:, :16].T, Wv[:, 144:].T
    Wc1T = Wc1.T
    bc11 = bc1[None, :]
    WcB = jnp.zeros((128, 128), f32)
    for cc in range(C):
        for dd in range(CD):
            WcB = WcB.at[:, 3 * cc + dd].set(Wc2[cc, :])
    ML = jnp.asarray(_ML_NP)
    MR = jnp.asarray(_MR_NP)

    hq, hkv = pl.pallas_call(
        _k1_body,
        out_shape=[jax.ShapeDtypeStruct((N, 128), f32),
                   jax.ShapeDtypeStruct((N, 256), f32)],
    )(h, WqT, WkvT, bq1, bkv1)

    k2 = pl.kernel(
        _k2_body,
        out_type=[jax.ShapeDtypeStruct((E, 128), f32),
                  jax.ShapeDtypeStruct((E, 128), f32),
                  jax.ShapeDtypeStruct((E, 256), f32)],
        mesh=plsc.VectorSubcoreMesh(**_SC_MESH),
        compiler_params=pltpu.CompilerParams(needs_layout_passes=False),
        scratch_types=[pltpu.VMEM((CB2,), jnp.int32),
                       pltpu.VMEM((CB2,), jnp.int32),
                       pltpu.VMEM((CB2, 128), f32),
                       pltpu.VMEM((CB2, 128), f32),
                       pltpu.VMEM((CB2, 128), f32),
                       pltpu.VMEM((CB2, 128), f32),
                       pltpu.VMEM((CB2, 256), f32),
                       pltpu.SemaphoreType.DMA],
    )
    cd128, hqr, hkvc = k2(row, col, ctab, hq, hkv)

    cdrad, ss = pl.pallas_call(
        _k3_body,
        grid=(E // EB3,),
        in_specs=[pl.BlockSpec((EB3, 128), lambda i: (i, 0)),
                  pl.BlockSpec((128, 48), lambda i: (0, 0)),
                  pl.BlockSpec((128, 48), lambda i: (0, 0))],
        out_specs=[pl.BlockSpec((EB3, 128), lambda i: (i, 0)),
                   pl.BlockSpec((1, 16), lambda i: (0, 0))],
        out_shape=[jax.ShapeDtypeStruct((E, 128), f32),
                   jax.ShapeDtypeStruct((1, 16), f32)],
    )(cd128, ML, MR)

    exv, trans = pl.pallas_call(
        _k4_body,
        grid=(E // EB4,),
        in_specs=[pl.BlockSpec((EB4, 128), lambda i: (i, 0)),
                  pl.BlockSpec((EB4, 16), lambda i: (i, 0)),
                  pl.BlockSpec((EB4, 128), lambda i: (i, 0)),
                  pl.BlockSpec((EB4, 256), lambda i: (i, 0)),
                  pl.BlockSpec((1, 16), lambda i: (0, 0)),
                  pl.BlockSpec((16, 128), lambda i: (0, 0)),
                  pl.BlockSpec((16, 128), lambda i: (0, 0)),
                  pl.BlockSpec((16, 128), lambda i: (0, 0)),
                  pl.BlockSpec((16, 128), lambda i: (0, 0)),
                  pl.BlockSpec((128, 128), lambda i: (0, 0)),
                  pl.BlockSpec((1, 128), lambda i: (0, 0)),
                  pl.BlockSpec((128, 128), lambda i: (0, 0))],
        out_specs=[pl.BlockSpec((EB4, 128), lambda i: (i, 0)),
                   pl.BlockSpec((EB4, 128), lambda i: (i, 0))],
        out_shape=[jax.ShapeDtypeStruct((E, 128), f32),
                   jax.ShapeDtypeStruct((E, 128), f32)],
    )(cdrad, edge_attr, hqr, hkvc, ss,
      WrkT, WrvT, WekT, WevT, Wc1T, bc11, WcB)

    zu = jnp.zeros((NP, 128), f32)

    def make_k5():
        return pl.kernel(
            _k5_body,
            out_type=[jax.ShapeDtypeStruct((2 * NP, 128), f32)],
            mesh=plsc.VectorSubcoreMesh(**_SC_MESH),
            compiler_params=pltpu.CompilerParams(needs_layout_passes=False),
            scratch_types=[pltpu.VMEM_SHARED((NP, 128), f32),
                           pltpu.VMEM((CB5,), jnp.int32),
                           pltpu.VMEM((CB5, 128), f32)],
        )

    (pu,) = make_k5()(row, exv, zu)
    (pc,) = make_k5()(row, trans, zu)

    h_out, c16o = pl.pallas_call(
        _k6_body,
        out_shape=[jax.ShapeDtypeStruct((N, 128), f32),
                   jax.ShapeDtypeStruct((N, 16), f32)],
    )(h, coord16, pu, pc)

    den1d = pc[0:N, 15] + pc[NP:NP + N, 15]
    k7 = pl.kernel(
        _k7_body,
        out_type=[jax.ShapeDtypeStruct((E,), f32)],
        mesh=plsc.VectorSubcoreMesh(**_SC_MESH),
        compiler_params=pltpu.CompilerParams(needs_layout_passes=False),
        scratch_types=[pltpu.VMEM((N,), f32),
                       pltpu.VMEM((CB7,), jnp.int32),
                       pltpu.VMEM((CB7, 128), f32),
                       pltpu.VMEM((CB7,), f32)],
    )
    (att,) = k7(row, trans, den1d)

    coord_out = c16o[:, :12].reshape(N, C, CD)
    return h_out, coord_out, att


# speedup vs baseline: 1.2059x; 1.0510x over previous
import numpy as np
import jax
import jax.numpy as jnp
from jax import lax
from jax.experimental import pallas as pl
from jax.experimental.pallas import tpu as pltpu
from jax.experimental.pallas import tpu_sc as plsc

N = 10000
E = 320000
C = 4
CD = 3

NC, NS, L = 2, 16, 16
NW = NC * NS
EPW = E // NW

CB2 = 80
CB5 = 80
NP = 10240
DCH = 80
DRND = NP // (DCH * NS)
CB7 = 400

EB3 = 2000
EB4 = 2000


def _radial_mats():
    ML = np.zeros((128, 48), np.float32)
    MR = np.zeros((128, 48), np.float32)
    for g in range(CD):
        for c in range(C):
            for f in range(C):
                ML[3 * c + g, g * 16 + c * 4 + f] = 1.0
                MR[3 * f + g, g * 16 + c * 4 + f] = 1.0
    return ML, MR


_ML_NP, _MR_NP = _radial_mats()


def _k1_body(h_ref, wq_ref, wkv_ref, bq_ref, bkv_ref, hq_ref, hkv_ref):
    h = h_ref[...]
    hq_ref[...] = jnp.dot(h, wq_ref[...], preferred_element_type=jnp.float32) + bq_ref[...]
    hkv_ref[...] = jnp.dot(h, wkv_ref[...], preferred_element_type=jnp.float32) + bkv_ref[...]


def _k2_body(row_hbm, col_hbm, ctab_hbm, hq_hbm, hkv_hbm,
             cd_hbm, hqr_hbm, hkvc_hbm,
             rowv, colv, ca, cb, cdv, qv, kvv, sem):
    c = lax.axis_index("c")
    s = lax.axis_index("s")
    wid = s * NC + c
    base = wid * EPW

    zero = jnp.zeros((L,), jnp.float32)

    def z8(j, carry):
        for kk in range(8):
            cdv[j, pl.ds(kk * L, L)] = zero
        return carry

    lax.fori_loop(0, CB2, z8, 0)

    def chunk(i, carry):
        off = base + i * CB2
        pltpu.sync_copy(row_hbm.at[pl.ds(off, CB2)], rowv)
        pltpu.sync_copy(col_hbm.at[pl.ds(off, CB2)], colv)
        cp1 = pltpu.async_copy(ctab_hbm.at[rowv], ca, sem)
        cp2 = pltpu.async_copy(ctab_hbm.at[colv], cb, sem)
        cp3 = pltpu.async_copy(hq_hbm.at[rowv], qv, sem)
        cp4 = pltpu.async_copy(hkv_hbm.at[colv], kvv, sem)
        cp1.wait()
        cp2.wait()
        cp3.wait()
        cp4.wait()

        def sub(j, carry2):
            cdv[j, pl.ds(0, L)] = ca[j, pl.ds(0, L)] - cb[j, pl.ds(0, L)]
            return carry2

        lax.fori_loop(0, CB2, sub, 0)
        pltpu.sync_copy(cdv, cd_hbm.at[pl.ds(off, CB2)])
        pltpu.sync_copy(qv, hqr_hbm.at[pl.ds(off, CB2)])
        pltpu.sync_copy(kvv, hkvc_hbm.at[pl.ds(off, CB2)])
        return carry

    lax.fori_loop(0, EPW // CB2, chunk, 0)


def _k3_body(cd_ref, ml_ref, mr_ref, cdrad_ref, ss_ref):
    cd = cd_ref[...]
    lm = jnp.dot(cd, ml_ref[...], preferred_element_type=jnp.float32)
    rm = jnp.dot(cd, mr_ref[...], preferred_element_type=jnp.float32)
    rad = (lm[:, 0:16] * rm[:, 0:16] + lm[:, 16:32] * rm[:, 16:32]
           + lm[:, 32:48] * rm[:, 32:48])
    cdrad_ref[...] = jnp.concatenate(
        [cd[:, 0:16], rad, cd[:, 32:128]], axis=1)

    @pl.when(pl.program_id(0) == 0)
    def _():
        ss_ref[...] = jnp.zeros_like(ss_ref)

    ss_ref[...] += jnp.sum(rad * rad, axis=0, keepdims=True)


def _k4_body(cdrad_ref, ea_ref, hqr_ref, hkvc_ref, ss_ref,
             wrk_ref, wrv_ref, wek_ref, wev_ref, wc1_ref, bc1_ref, wcb_ref,
             exv_ref, tr_ref):
    inv = 1.0 / jnp.maximum(jnp.sqrt(ss_ref[...]), 1e-12)
    cdrad = cdrad_ref[...]
    radn = cdrad[:, 16:32] * inv
    ea = ea_ref[...]
    k = (hkvc_ref[:, :128]
         + jnp.dot(radn, wrk_ref[...], preferred_element_type=jnp.float32)
         + jnp.dot(ea, wek_ref[...], preferred_element_type=jnp.float32))
    v = (hkvc_ref[:, 128:]
         + jnp.dot(radn, wrv_ref[...], preferred_element_type=jnp.float32)
         + jnp.dot(ea, wev_ref[...], preferred_element_type=jnp.float32))
    alpha = jnp.sum(hqr_ref[...] * k, axis=1, keepdims=True)
    ex = jnp.exp(alpha)
    exv_ref[...] = ex * v
    u = jnp.dot(v, wc1_ref[...], preferred_element_type=jnp.float32) + bc1_ref[...]
    su = u * jax.nn.sigmoid(u)
    cmb = jnp.dot(su, wcb_ref[...], preferred_element_type=jnp.float32)
    tr = cdrad * (ex * cmb)
    lane = lax.broadcasted_iota(jnp.int32, tr.shape, 1)
    tr = jnp.where(lane >= 3 * C, 0.0, tr)
    tr = jnp.where(lane == 15, ex, tr)
    tr_ref[...] = tr


def _k5_body(row_hbm, x_hbm, z_hbm, p_hbm, agg, idxv, dbuf):
    c = lax.axis_index("c")
    s = lax.axis_index("s")
    wid = s * NC + c

    def zinit(k, carry):
        r = (k * NS + s) * DCH
        pltpu.sync_copy(z_hbm.at[pl.ds(r, DCH)], dbuf)
        pltpu.sync_copy(dbuf, agg.at[pl.ds(r, DCH)])
        return carry

    lax.fori_loop(0, DRND, zinit, 0)
    plsc.subcore_barrier()

    base = wid * EPW

    def chunk(i, carry):
        off = base + i * CB5
        pltpu.sync_copy(row_hbm.at[pl.ds(off, CB5)], idxv)
        pltpu.sync_copy(x_hbm.at[pl.ds(off, CB5)], dbuf)
        pltpu.sync_copy(dbuf, agg.at[idxv], add=True)
        return carry

    lax.fori_loop(0, EPW // CB5, chunk, 0)
    plsc.subcore_barrier()

    def dump(k, carry):
        r = (k * NS + s) * DCH
        pltpu.sync_copy(agg.at[pl.ds(r, DCH)], dbuf)
        pltpu.sync_copy(dbuf, p_hbm.at[pl.ds(c * NP + r, DCH)])
        return carry

    lax.fori_loop(0, DRND, dump, 0)


def _k6_body(h_ref, c16_ref, pu_ref, pc_ref, hout_ref, cout_ref):
    aggc = pc_ref[0:N, 0:16] + pc_ref[NP:NP + N, 0:16]
    den = aggc[:, 15:16]
    rden = jnp.where(den != 0.0, 1.0 / den, 0.0)
    hout_ref[...] = h_ref[...] + (pu_ref[0:N, :] + pu_ref[NP:NP + N, :]) * rden
    cout_ref[...] = c16_ref[...] + aggc * rden


def _k7_body(row_hbm, tr_hbm, den_hbm, att_hbm, denv, idxv, trv, av):
    c = lax.axis_index("c")
    s = lax.axis_index("s")
    wid = s * NC + c
    base = wid * EPW
    l15 = jnp.full((L,), 15, jnp.int32)
    pltpu.sync_copy(den_hbm, denv)

    def chunk(i, carry):
        off = base + i * CB7
        pltpu.sync_copy(row_hbm.at[pl.ds(off, CB7)], idxv)
        pltpu.sync_copy(tr_hbm.at[pl.ds(off, CB7)], trv)

        def j16(j, carry2):
            eidx = idxv[pl.ds(j * L, L)]
            den = plsc.load_gather(denv, [eidx])
            ridx = j * L + lax.iota(jnp.int32, L)
            ex = plsc.load_gather(trv, [ridx, l15])
            av[pl.ds(j * L, L)] = ex / den
            return carry2

        lax.fori_loop(0, CB7 // L, j16, 0)
        pltpu.sync_copy(av, att_hbm.at[pl.ds(off, CB7)])
        return carry

    lax.fori_loop(0, EPW // CB7, chunk, 0)


_SC_MESH = dict(core_axis_name="c", subcore_axis_name="s",
                num_cores=NC, num_subcores=NS)


def kernel(h, edge_index, coord, edge_attr, Wq, bq, Wkv, bkv, Wc1, bc1, Wc2):
    f32 = jnp.float32
    row = edge_index[0]
    col = edge_index[1]
    ctab = jnp.pad(coord.reshape(N, C * CD), ((0, 0), (0, 128 - C * CD)))
    coord16 = ctab[:, :16]

    Wk = Wkv[0::2]
    Wv = Wkv[1::2]
    bk = bkv[0::2]
    bv = bkv[1::2]
    WqT = Wq.T
    WkvT = jnp.concatenate([Wk[:, 16:144].T, Wv[:, 16:144].T], axis=1)
    bkv1 = jnp.concatenate([bk, bv])[None, :]
    bq1 = bq[None, :]
    WrkT, WekT = Wk[:, :16].T, Wk[:, 144:].T
    WrvT, WevT = Wv[:, :16].T, Wv[:, 144:].T
    Wc1T = Wc1.T
    bc11 = bc1[None, :]
    WcB = jnp.zeros((128, 128), f32)
    for cc in range(C):
        for dd in range(CD):
            WcB = WcB.at[:, 3 * cc + dd].set(Wc2[cc, :])
    ML = jnp.asarray(_ML_NP)
    MR = jnp.asarray(_MR_NP)

    hq, hkv = pl.pallas_call(
        _k1_body,
        out_shape=[jax.ShapeDtypeStruct((N, 128), f32),
                   jax.ShapeDtypeStruct((N, 256), f32)],
    )(h, WqT, WkvT, bq1, bkv1)

    k2 = pl.kernel(
        _k2_body,
        out_type=[jax.ShapeDtypeStruct((E, 128), f32),
                  jax.ShapeDtypeStruct((E, 128), f32),
                  jax.ShapeDtypeStruct((E, 256), f32)],
        mesh=plsc.VectorSubcoreMesh(**_SC_MESH),
        compiler_params=pltpu.CompilerParams(needs_layout_passes=False),
        scratch_types=[pltpu.VMEM((CB2,), jnp.int32),
                       pltpu.VMEM((CB2,), jnp.int32),
                       pltpu.VMEM((CB2, 128), f32),
                       pltpu.VMEM((CB2, 128), f32),
                       pltpu.VMEM((CB2, 128), f32),
                       pltpu.VMEM((CB2, 128), f32),
                       pltpu.VMEM((CB2, 256), f32),
                       pltpu.SemaphoreType.DMA],
    )
    cd128, hqr, hkvc = k2(row, col, ctab, hq, hkv)

    cdrad, ss = pl.pallas_call(
        _k3_body,
        grid=(E // EB3,),
        in_specs=[pl.BlockSpec((EB3, 128), lambda i: (i, 0)),
                  pl.BlockSpec((128, 48), lambda i: (0, 0)),
                  pl.BlockSpec((128, 48), lambda i: (0, 0))],
        out_specs=[pl.BlockSpec((EB3, 128), lambda i: (i, 0)),
                   pl.BlockSpec((1, 16), lambda i: (0, 0))],
        out_shape=[jax.ShapeDtypeStruct((E, 128), f32),
                   jax.ShapeDtypeStruct((1, 16), f32)],
    )(cd128, ML, MR)

    exv, trans = pl.pallas_call(
        _k4_body,
        grid=(E // EB4,),
        in_specs=[pl.BlockSpec((EB4, 128), lambda i: (i, 0)),
                  pl.BlockSpec((EB4, 16), lambda i: (i, 0)),
                  pl.BlockSpec((EB4, 128), lambda i: (i, 0)),
                  pl.BlockSpec((EB4, 256), lambda i: (i, 0)),
                  pl.BlockSpec((1, 16), lambda i: (0, 0)),
                  pl.BlockSpec((16, 128), lambda i: (0, 0)),
                  pl.BlockSpec((16, 128), lambda i: (0, 0)),
                  pl.BlockSpec((16, 128), lambda i: (0, 0)),
                  pl.BlockSpec((16, 128), lambda i: (0, 0)),
                  pl.BlockSpec((128, 128), lambda i: (0, 0)),
                  pl.BlockSpec((1, 128), lambda i: (0, 0)),
                  pl.BlockSpec((128, 128), lambda i: (0, 0))],
        out_specs=[pl.BlockSpec((EB4, 128), lambda i: (i, 0)),
                   pl.BlockSpec((EB4, 128), lambda i: (i, 0))],
        out_shape=[jax.ShapeDtypeStruct((E, 128), f32),
                   jax.ShapeDtypeStruct((E, 128), f32)],
    )(cdrad, edge_attr, hqr, hkvc, ss,
      WrkT, WrvT, WekT, WevT, Wc1T, bc11, WcB)

    zu = jnp.zeros((NP, 128), f32)

    def make_k5():
        return pl.kernel(
            _k5_body,
            out_type=[jax.ShapeDtypeStruct((2 * NP, 128), f32)],
            mesh=plsc.VectorSubcoreMesh(**_SC_MESH),
            compiler_params=pltpu.CompilerParams(needs_layout_passes=False),
            scratch_types=[pltpu.VMEM_SHARED((NP, 128), f32),
                           pltpu.VMEM((CB5,), jnp.int32),
                           pltpu.VMEM((CB5, 128), f32)],
        )

    (pu,) = make_k5()(row, exv, zu)
    (pc,) = make_k5()(row, trans, zu)

    h_out, c16o = pl.pallas_call(
        _k6_body,
        out_shape=[jax.ShapeDtypeStruct((N, 128), f32),
                   jax.ShapeDtypeStruct((N, 16), f32)],
    )(h, coord16, pu, pc)

    den1d = pc[0:N, 15] + pc[NP:NP + N, 15]
    k7 = pl.kernel(
        _k7_body,
        out_type=[jax.ShapeDtypeStruct((E,), f32)],
        mesh=plsc.VectorSubcoreMesh(**_SC_MESH),
        compiler_params=pltpu.CompilerParams(needs_layout_passes=False),
        scratch_types=[pltpu.VMEM((N,), f32),
                       pltpu.VMEM((CB7,), jnp.int32),
                       pltpu.VMEM((CB7, 128), f32),
                       pltpu.VMEM((CB7,), f32)],
    )
    (att,) = k7(row, trans, den1d)

    coord_out = c16o[:, :12].reshape(N, C, CD)
    return h_out, coord_out, att
